# Initial kernel scaffold; baseline (speedup 1.0000x reference)
import jax
import jax.numpy as jnp
from jax.experimental import pallas as pl


def kernel(xyz, norm):
    raise NotImplementedError("write your pallas kernel here")


# speedup vs baseline: 2.8169x; 2.8169x over previous
import functools

import jax
import jax.numpy as jnp
from jax import lax
from jax.experimental import pallas as pl
from jax.experimental.pallas import tpu as pltpu
from jax.experimental.pallas import tpu_sc as plsc

B = 8
N = 8192
G = 128
K = 32
L = 16
NCH = N // L
QC = G // 4
CAND_CAP = 2048

_mesh = plsc.VectorSubcoreMesh(core_axis_name="c", subcore_axis_name="s")

_KERNEL_KWARGS = dict(
    out_type=[
        jax.ShapeDtypeStruct((B, G, K, 3), jnp.float32),
        jax.ShapeDtypeStruct((B, G, 3), jnp.float32),
        jax.ShapeDtypeStruct((B, G, K, 3), jnp.float32),
        jax.ShapeDtypeStruct((B, G, 3), jnp.float32),
        jax.ShapeDtypeStruct((B, G, K), jnp.int32),
        jax.ShapeDtypeStruct((B, G), jnp.int32),
    ],
    mesh=_mesh,
    compiler_params=pltpu.CompilerParams(
        needs_layout_passes=False, use_tc_tiling_on_sc=False),
    scratch_types=[
        pltpu.VMEM((N,), jnp.float32),
        pltpu.VMEM((N,), jnp.float32),
        pltpu.VMEM((N,), jnp.float32),
        pltpu.VMEM((N,), jnp.float32),
        pltpu.VMEM((N,), jnp.float32),
        pltpu.VMEM((N,), jnp.float32),
        pltpu.VMEM((N,), jnp.float32),
        pltpu.VMEM((N,), jnp.float32),
        pltpu.VMEM((N,), jnp.float32),
        pltpu.VMEM((N,), jnp.float32),
        pltpu.VMEM((N,), jnp.float32),
        pltpu.VMEM((N,), jnp.float32),
        pltpu.VMEM((G,), jnp.int32),
        pltpu.VMEM((CAND_CAP + L,), jnp.float32),
        pltpu.VMEM((CAND_CAP + L,), jnp.int32),
        pltpu.VMEM((QC, K), jnp.int32),
        pltpu.VMEM((QC, K, 3), jnp.float32),
        pltpu.VMEM((QC, K, 3), jnp.float32),
        pltpu.VMEM((QC, 3), jnp.float32),
        pltpu.VMEM((QC, 3), jnp.float32),
        pltpu.VMEM((QC,), jnp.float32),
        pltpu.VMEM((QC,), jnp.float32),
        pltpu.VMEM((QC,), jnp.float32),
        pltpu.VMEM((QC,), jnp.float32),
    ],
)


def _group_sc_body(xyz_hbm, norm_hbm,
              nb_out, ctr_out, ng_out, nctr_out, idx_out, idxnew_out,
              xv, yv, zv, nxv, nyv, nzv, xbv, ybv, zbv, xx, dist, d2buf, cent,
              candd, candi, selidx,
              nb_stage, ng_stage, ctr_stage, nctr_stage,
              ccx, ccy, ccz, ccsq):
    wid = lax.axis_index("s") * 2 + lax.axis_index("c")
    b = wid % B
    q = wid // B

    lane = lax.broadcasted_iota(jnp.int32, (L,), 0)
    lane0 = lane == 0
    zeros16 = jnp.zeros((L,), jnp.int32)
    ones16 = jnp.full((L,), 1, jnp.int32)
    twos16 = jnp.full((L,), 2, jnp.int32)
    inf16 = jnp.full((L,), jnp.inf, jnp.float32)

    pltpu.sync_copy(xyz_hbm.at[b, 0], xv)
    pltpu.sync_copy(xyz_hbm.at[b, 1], yv)
    pltpu.sync_copy(xyz_hbm.at[b, 2], zv)
    pltpu.sync_copy(norm_hbm.at[b, 0], nxv)
    pltpu.sync_copy(norm_hbm.at[b, 1], nyv)
    pltpu.sync_copy(norm_hbm.at[b, 2], nzv)

    def _bf16r(v):
        u = plsc.bitcast(v, jnp.int32)
        r = (u + jnp.int32(0x7FFF)) + ((u >> 16) & jnp.int32(1))
        return plsc.bitcast(r & jnp.int32(-65536), jnp.float32)

    def _pre(j, c):
        s = pl.ds(j * L, L)
        x = xv[s]
        y = yv[s]
        z = zv[s]
        xx[s] = x * x + y * y + z * z
        xbv[s] = _bf16r(x)
        ybv[s] = _bf16r(y)
        zbv[s] = _bf16r(z)
        dist[s] = jnp.full((L,), 1e10, jnp.float32)
        return c
    lax.fori_loop(0, NCH, _pre, 0)

    def _fps_step(i, f):
        fv = jnp.full((L,), f, jnp.int32)
        plsc.store_scatter(cent, [jnp.full((L,), i, jnp.int32)], fv,
                           mask=lane0)
        cx = plsc.load_gather(xv, [fv])
        cy = plsc.load_gather(yv, [fv])
        cz = plsc.load_gather(zv, [fv])

        def _chunk(j, carry):
            vmax, vidx = carry
            s = pl.ds(j * L, L)
            dx = xv[s] - cx
            dy = yv[s] - cy
            dz = zv[s] - cz
            d = dx * dx + dy * dy + dz * dz
            nd = jnp.minimum(dist[s], d)
            dist[s] = nd
            gt = nd > vmax
            vidx = jnp.where(gt, j * L + lane, vidx)
            vmax = jnp.where(gt, nd, vmax)
            return vmax, vidx

        vmax, vidx = lax.fori_loop(
            0, NCH, _chunk,
            (jnp.full((L,), -1.0, jnp.float32), zeros16))
        m = jnp.max(vmax)
        cand = jnp.where(vmax == m, vidx, jnp.full((L,), N, jnp.int32))
        return jnp.min(cand)

    lax.fori_loop(0, G, _fps_step, jnp.int32(0))

    for h in (0, 1):
        cidx = plsc.load_gather(cent, [q * QC + h * L + lane])
        rows = h * L + lane
        sx = plsc.load_gather(xv, [cidx])
        sy = plsc.load_gather(yv, [cidx])
        sz = plsc.load_gather(zv, [cidx])
        ccx[pl.ds(h * L, L)] = sx
        ccy[pl.ds(h * L, L)] = sy
        ccz[pl.ds(h * L, L)] = sz
        ccsq[pl.ds(h * L, L)] = plsc.load_gather(xx, [cidx])
        plsc.store_scatter(ctr_stage, [rows, zeros16], sx)
        plsc.store_scatter(ctr_stage, [rows, ones16], sy)
        plsc.store_scatter(ctr_stage, [rows, twos16], sz)
        plsc.store_scatter(nctr_stage, [rows, zeros16],
                           plsc.load_gather(nxv, [cidx]))
        plsc.store_scatter(nctr_stage, [rows, ones16],
                           plsc.load_gather(nyv, [cidx]))
        plsc.store_scatter(nctr_stage, [rows, twos16],
                           plsc.load_gather(nzv, [cidx]))

    def _centroid(gl, c):
        glv = jnp.full((L,), gl, jnp.int32)
        cxg = _bf16r(plsc.load_gather(ccx, [glv]))
        cyg = _bf16r(plsc.load_gather(ccy, [glv]))
        czg = _bf16r(plsc.load_gather(ccz, [glv]))
        ccg = plsc.load_gather(ccsq, [glv])

        def _p1(j, carry):
            vm1, vm2 = carry
            s = pl.ds(j * L, L)
            t = (cxg * xbv[s] + cyg * ybv[s]) + czg * zbv[s]
            d2 = jnp.maximum((ccg + xx[s]) - 2.0 * t, 0.0)
            d2buf[s] = d2
            isnew = d2 < vm1
            vm2 = jnp.where(isnew, vm1, jnp.minimum(vm2, d2))
            vm1 = jnp.minimum(vm1, d2)
            return vm1, vm2
        vm1, vm2 = lax.fori_loop(0, NCH, _p1, (inf16, inf16))
        ubound = jnp.max(vm2)

        def _p2(j, cnt):
            s = pl.ds(j * L, L)
            d2 = d2buf[s]
            msk = d2 <= ubound
            pos = cnt + plsc.cumsum(msk.astype(jnp.int32)) - 1
            plsc.store_scatter(candd, [pos], d2, mask=msk)
            plsc.store_scatter(candi, [pos], j * L + lane, mask=msk)
            return cnt + plsc.all_reduce_population_count(msk)
        cntv = lax.fori_loop(0, NCH, _p2, zeros16)
        m_total = jnp.max(cntv)
        plsc.store_scatter(candd, [m_total + lane], inf16)
        nch_c = lax.div(m_total + (L - 1), L)

        def _sel(k, c2):
            def _scan(j, carry):
                vmin, vpos = carry
                s = pl.ds(j * L, L)
                d = candd[s]
                lt = d < vmin
                vpos = jnp.where(lt, j * L + lane, vpos)
                vmin = jnp.where(lt, d, vmin)
                return vmin, vpos
            vmin, vpos = lax.fori_loop(0, nch_c, _scan, (inf16, zeros16))
            mv = jnp.min(vmin)
            pc = jnp.where(vmin == mv, vpos,
                           jnp.full((L,), CAND_CAP + L, jnp.int32))
            p = jnp.min(pc)
            pv = jnp.full((L,), p, jnp.int32)
            iv = plsc.load_gather(candi, [pv])
            plsc.store_scatter(selidx, [glv, jnp.full((L,), k, jnp.int32)],
                               iv, mask=lane0)
            plsc.store_scatter(candd, [pv], inf16, mask=lane0)
            return c2
        lax.fori_loop(0, K, _sel, 0)
        return c
    lax.fori_loop(0, QC, _centroid, 0)

    def _gather(gl, c):
        glv = jnp.full((L,), gl, jnp.int32)
        cxg = plsc.load_gather(ccx, [glv])
        cyg = plsc.load_gather(ccy, [glv])
        czg = plsc.load_gather(ccz, [glv])

        for h in (0, 1):
            ks = h * L + lane
            ii = plsc.load_gather(selidx, [glv, ks])
            gx = plsc.load_gather(xv, [ii])
            gy = plsc.load_gather(yv, [ii])
            gz = plsc.load_gather(zv, [ii])
            plsc.store_scatter(nb_stage, [glv, ks, zeros16], gx - cxg)
            plsc.store_scatter(nb_stage, [glv, ks, ones16], gy - cyg)
            plsc.store_scatter(nb_stage, [glv, ks, twos16], gz - czg)
            plsc.store_scatter(ng_stage, [glv, ks, zeros16],
                               plsc.load_gather(nxv, [ii]))
            plsc.store_scatter(ng_stage, [glv, ks, ones16],
                               plsc.load_gather(nyv, [ii]))
            plsc.store_scatter(ng_stage, [glv, ks, twos16],
                               plsc.load_gather(nzv, [ii]))
        return c
    lax.fori_loop(0, QC, _gather, 0)

    gsl = pl.ds(q * QC, QC)
    pltpu.sync_copy(nb_stage, nb_out.at[b, gsl])
    pltpu.sync_copy(ctr_stage, ctr_out.at[b, gsl])
    pltpu.sync_copy(ng_stage, ng_out.at[b, gsl])
    pltpu.sync_copy(nctr_stage, nctr_out.at[b, gsl])
    pltpu.sync_copy(selidx, idx_out.at[b, gsl])
    pltpu.sync_copy(cent.at[gsl], idxnew_out.at[b, gsl])


_group_sc = pl.kernel(_group_sc_body, **_KERNEL_KWARGS)


def kernel(xyz, norm):
    xyz_t = jnp.transpose(xyz, (0, 2, 1))
    norm_t = jnp.transpose(norm, (0, 2, 1))
    nb, ctr, ng, nctr, idx, idx_new = _group_sc(xyz_t, norm_t)
    return (nb, ctr, ng, nctr, idx, idx_new)


# speedup vs baseline: 7.4506x; 2.6449x over previous
import functools

import jax
import jax.numpy as jnp
from jax import lax
from jax.experimental import pallas as pl
from jax.experimental.pallas import tpu as pltpu
from jax.experimental.pallas import tpu_sc as plsc

B = 8
N = 8192
G = 128
K = 32
L = 16
NCH = N // L
QC = G // 4
CAND_CAP = 2048

_mesh = plsc.VectorSubcoreMesh(core_axis_name="c", subcore_axis_name="s")

_KERNEL_KWARGS = dict(
    out_type=[
        jax.ShapeDtypeStruct((B, G, K, 3), jnp.float32),
        jax.ShapeDtypeStruct((B, G, 3), jnp.float32),
        jax.ShapeDtypeStruct((B, G, K, 3), jnp.float32),
        jax.ShapeDtypeStruct((B, G, 3), jnp.float32),
        jax.ShapeDtypeStruct((B, G, K), jnp.int32),
        jax.ShapeDtypeStruct((B, G), jnp.int32),
    ],
    mesh=_mesh,
    compiler_params=pltpu.CompilerParams(
        needs_layout_passes=False, use_tc_tiling_on_sc=False),
    scratch_types=[
        pltpu.VMEM((N,), jnp.float32),
        pltpu.VMEM((N,), jnp.float32),
        pltpu.VMEM((N,), jnp.float32),
        pltpu.VMEM((N,), jnp.float32),
        pltpu.VMEM((N,), jnp.float32),
        pltpu.VMEM((N,), jnp.float32),
        pltpu.VMEM((N,), jnp.float32),
        pltpu.VMEM((N,), jnp.float32),
        pltpu.VMEM((N,), jnp.float32),
        pltpu.VMEM((N,), jnp.float32),
        pltpu.VMEM((N,), jnp.float32),
        pltpu.VMEM((N,), jnp.float32),
        pltpu.VMEM((G,), jnp.int32),
        pltpu.VMEM((CAND_CAP + L,), jnp.float32),
        pltpu.VMEM((CAND_CAP + L,), jnp.int32),
        pltpu.VMEM((QC, K), jnp.int32),
        pltpu.VMEM((QC, K, 3), jnp.float32),
        pltpu.VMEM((QC, K, 3), jnp.float32),
        pltpu.VMEM((QC, 3), jnp.float32),
        pltpu.VMEM((QC, 3), jnp.float32),
        pltpu.VMEM((QC,), jnp.float32),
        pltpu.VMEM((QC,), jnp.float32),
        pltpu.VMEM((QC,), jnp.float32),
        pltpu.VMEM((QC,), jnp.float32),
    ],
)


def _group_sc_body(xyz_hbm, norm_hbm,
              nb_out, ctr_out, ng_out, nctr_out, idx_out, idxnew_out,
              xv, yv, zv, nxv, nyv, nzv, xbv, ybv, zbv, xx, dist, d2buf, cent,
              candd, candi, selidx,
              nb_stage, ng_stage, ctr_stage, nctr_stage,
              ccx, ccy, ccz, ccsq):
    wid = lax.axis_index("s") * 2 + lax.axis_index("c")
    b = wid % B
    q = wid // B

    lane = lax.broadcasted_iota(jnp.int32, (L,), 0)
    lane0 = lane == 0
    zeros16 = jnp.zeros((L,), jnp.int32)
    ones16 = jnp.full((L,), 1, jnp.int32)
    twos16 = jnp.full((L,), 2, jnp.int32)
    inf16 = jnp.full((L,), jnp.inf, jnp.float32)

    pltpu.sync_copy(xyz_hbm.at[b, 0], xv)
    pltpu.sync_copy(xyz_hbm.at[b, 1], yv)
    pltpu.sync_copy(xyz_hbm.at[b, 2], zv)
    pltpu.sync_copy(norm_hbm.at[b, 0], nxv)
    pltpu.sync_copy(norm_hbm.at[b, 1], nyv)
    pltpu.sync_copy(norm_hbm.at[b, 2], nzv)

    def _bf16r(v):
        u = plsc.bitcast(v, jnp.int32)
        r = (u + jnp.int32(0x7FFF)) + ((u >> 16) & jnp.int32(1))
        return plsc.bitcast(r & jnp.int32(-65536), jnp.float32)

    @plsc.parallel_loop(0, N, step=L, unroll=8)
    def _pre(i):
        s = pl.ds(i, L)
        x = xv[s]
        y = yv[s]
        z = zv[s]
        xx[s] = x * x + y * y + z * z
        xbv[s] = _bf16r(x)
        ybv[s] = _bf16r(y)
        zbv[s] = _bf16r(z)
        dist[s] = jnp.full((L,), 1e10, jnp.float32)

    def _fps_step(i, f):
        fv = jnp.full((L,), f, jnp.int32)
        plsc.store_scatter(cent, [jnp.full((L,), i, jnp.int32)], fv,
                           mask=lane0)
        cx = plsc.load_gather(xv, [fv])
        cy = plsc.load_gather(yv, [fv])
        cz = plsc.load_gather(zv, [fv])

        @plsc.parallel_loop(
            0, N, step=L, unroll=8,
            carry=(jnp.full((L,), -1.0, jnp.float32), zeros16))
        def _chunk(i, carry):
            vmax, vidx = carry
            s = pl.ds(i, L)
            dx = xv[s] - cx
            dy = yv[s] - cy
            dz = zv[s] - cz
            d = dx * dx + dy * dy + dz * dz
            nd = jnp.minimum(dist[s], d)
            dist[s] = nd
            gt = nd > vmax
            vidx = jnp.where(gt, i + lane, vidx)
            vmax = jnp.where(gt, nd, vmax)
            return vmax, vidx

        vmax, vidx = _chunk
        m = jnp.max(vmax)
        cand = jnp.where(vmax == m, vidx, jnp.full((L,), N, jnp.int32))
        return jnp.min(cand)

    lax.fori_loop(0, G, _fps_step, jnp.int32(0))

    for h in (0, 1):
        cidx = plsc.load_gather(cent, [q * QC + h * L + lane])
        rows = h * L + lane
        sx = plsc.load_gather(xv, [cidx])
        sy = plsc.load_gather(yv, [cidx])
        sz = plsc.load_gather(zv, [cidx])
        ccx[pl.ds(h * L, L)] = sx
        ccy[pl.ds(h * L, L)] = sy
        ccz[pl.ds(h * L, L)] = sz
        ccsq[pl.ds(h * L, L)] = plsc.load_gather(xx, [cidx])
        plsc.store_scatter(ctr_stage, [rows, zeros16], sx)
        plsc.store_scatter(ctr_stage, [rows, ones16], sy)
        plsc.store_scatter(ctr_stage, [rows, twos16], sz)
        plsc.store_scatter(nctr_stage, [rows, zeros16],
                           plsc.load_gather(nxv, [cidx]))
        plsc.store_scatter(nctr_stage, [rows, ones16],
                           plsc.load_gather(nyv, [cidx]))
        plsc.store_scatter(nctr_stage, [rows, twos16],
                           plsc.load_gather(nzv, [cidx]))

    def _centroid(gl, c):
        glv = jnp.full((L,), gl, jnp.int32)
        cxg = _bf16r(plsc.load_gather(ccx, [glv]))
        cyg = _bf16r(plsc.load_gather(ccy, [glv]))
        czg = _bf16r(plsc.load_gather(ccz, [glv]))
        ccg = plsc.load_gather(ccsq, [glv])

        @plsc.parallel_loop(0, N, step=L, unroll=8, carry=(inf16, inf16))
        def _p1(i, carry):
            vm1, vm2 = carry
            s = pl.ds(i, L)
            t = (cxg * xbv[s] + cyg * ybv[s]) + czg * zbv[s]
            d2 = jnp.maximum((ccg + xx[s]) - 2.0 * t, 0.0)
            d2buf[s] = d2
            isnew = d2 < vm1
            vm2 = jnp.where(isnew, vm1, jnp.minimum(vm2, d2))
            vm1 = jnp.minimum(vm1, d2)
            return vm1, vm2
        vm1, vm2 = _p1
        ubound = jnp.max(vm2)

        @plsc.parallel_loop(0, N, step=L, unroll=8, carry=zeros16)
        def _p2(i, cnt):
            s = pl.ds(i, L)
            d2 = d2buf[s]
            msk = d2 <= ubound
            pos = cnt + plsc.cumsum(msk.astype(jnp.int32)) - 1
            plsc.store_scatter(candd, [pos], d2, mask=msk)
            plsc.store_scatter(candi, [pos], i + lane, mask=msk)
            return cnt + plsc.all_reduce_population_count(msk)
        cntv = _p2
        m_total = jnp.max(cntv)
        plsc.store_scatter(candd, [m_total + lane], inf16)
        nch_c = lax.div(m_total + (L - 1), L)

        def _sel(k, c2):
            def _scan(j, carry):
                vmin, vpos = carry
                s = pl.ds(j * L, L)
                d = candd[s]
                lt = d < vmin
                vpos = jnp.where(lt, j * L + lane, vpos)
                vmin = jnp.where(lt, d, vmin)
                return vmin, vpos
            vmin, vpos = lax.fori_loop(0, nch_c, _scan, (inf16, zeros16))
            mv = jnp.min(vmin)
            pc = jnp.where(vmin == mv, vpos,
                           jnp.full((L,), CAND_CAP + L, jnp.int32))
            p = jnp.min(pc)
            pv = jnp.full((L,), p, jnp.int32)
            iv = plsc.load_gather(candi, [pv])
            plsc.store_scatter(selidx, [glv, jnp.full((L,), k, jnp.int32)],
                               iv, mask=lane0)
            plsc.store_scatter(candd, [pv], inf16, mask=lane0)
            return c2
        lax.fori_loop(0, K, _sel, 0)
        return c
    lax.fori_loop(0, QC, _centroid, 0)

    def _gather(gl, c):
        glv = jnp.full((L,), gl, jnp.int32)
        cxg = plsc.load_gather(ccx, [glv])
        cyg = plsc.load_gather(ccy, [glv])
        czg = plsc.load_gather(ccz, [glv])

        for h in (0, 1):
            ks = h * L + lane
            ii = plsc.load_gather(selidx, [glv, ks])
            gx = plsc.load_gather(xv, [ii])
            gy = plsc.load_gather(yv, [ii])
            gz = plsc.load_gather(zv, [ii])
            plsc.store_scatter(nb_stage, [glv, ks, zeros16], gx - cxg)
            plsc.store_scatter(nb_stage, [glv, ks, ones16], gy - cyg)
            plsc.store_scatter(nb_stage, [glv, ks, twos16], gz - czg)
            plsc.store_scatter(ng_stage, [glv, ks, zeros16],
                               plsc.load_gather(nxv, [ii]))
            plsc.store_scatter(ng_stage, [glv, ks, ones16],
                               plsc.load_gather(nyv, [ii]))
            plsc.store_scatter(ng_stage, [glv, ks, twos16],
                               plsc.load_gather(nzv, [ii]))
        return c
    lax.fori_loop(0, QC, _gather, 0)

    gsl = pl.ds(q * QC, QC)
    pltpu.sync_copy(nb_stage, nb_out.at[b, gsl])
    pltpu.sync_copy(ctr_stage, ctr_out.at[b, gsl])
    pltpu.sync_copy(ng_stage, ng_out.at[b, gsl])
    pltpu.sync_copy(nctr_stage, nctr_out.at[b, gsl])
    pltpu.sync_copy(selidx, idx_out.at[b, gsl])
    pltpu.sync_copy(cent.at[gsl], idxnew_out.at[b, gsl])


_group_sc = pl.kernel(_group_sc_body, **_KERNEL_KWARGS)


def kernel(xyz, norm):
    xyz_t = jnp.transpose(xyz, (0, 2, 1))
    norm_t = jnp.transpose(norm, (0, 2, 1))
    nb, ctr, ng, nctr, idx, idx_new = _group_sc(xyz_t, norm_t)
    return (nb, ctr, ng, nctr, idx, idx_new)


# speedup vs baseline: 9.8188x; 1.3178x over previous
import functools

import jax
import jax.numpy as jnp
from jax import lax
from jax.experimental import pallas as pl
from jax.experimental.pallas import tpu as pltpu
from jax.experimental.pallas import tpu_sc as plsc

B = 8
N = 8192
G = 128
K = 32
L = 16
NCH = N // L
QC = G // 4
CAND_CAP = 2048

_mesh = plsc.VectorSubcoreMesh(core_axis_name="c", subcore_axis_name="s")

_KERNEL_KWARGS = dict(
    out_type=[
        jax.ShapeDtypeStruct((B, G, K, 3), jnp.float32),
        jax.ShapeDtypeStruct((B, G, 3), jnp.float32),
        jax.ShapeDtypeStruct((B, G, K, 3), jnp.float32),
        jax.ShapeDtypeStruct((B, G, 3), jnp.float32),
        jax.ShapeDtypeStruct((B, G, K), jnp.int32),
        jax.ShapeDtypeStruct((B, G), jnp.int32),
    ],
    mesh=_mesh,
    compiler_params=pltpu.CompilerParams(
        needs_layout_passes=False, use_tc_tiling_on_sc=False),
    scratch_types=[
        pltpu.VMEM((N,), jnp.float32),
        pltpu.VMEM((N,), jnp.float32),
        pltpu.VMEM((N,), jnp.float32),
        pltpu.VMEM((N,), jnp.float32),
        pltpu.VMEM((N,), jnp.float32),
        pltpu.VMEM((N,), jnp.float32),
        pltpu.VMEM((N,), jnp.float32),
        pltpu.VMEM((N,), jnp.float32),
        pltpu.VMEM((N,), jnp.float32),
        pltpu.VMEM((N,), jnp.float32),
        pltpu.VMEM((N,), jnp.float32),
        pltpu.VMEM((N,), jnp.float32),
        pltpu.VMEM((G,), jnp.int32),
        pltpu.VMEM((CAND_CAP + L,), jnp.float32),
        pltpu.VMEM((CAND_CAP + L,), jnp.int32),
        pltpu.VMEM((QC, K), jnp.int32),
        pltpu.VMEM((QC, K, 3), jnp.float32),
        pltpu.VMEM((QC, K, 3), jnp.float32),
        pltpu.VMEM((QC, 3), jnp.float32),
        pltpu.VMEM((QC, 3), jnp.float32),
        pltpu.VMEM((QC,), jnp.float32),
        pltpu.VMEM((QC,), jnp.float32),
        pltpu.VMEM((QC,), jnp.float32),
        pltpu.VMEM((QC,), jnp.float32),
        pltpu.VMEM_SHARED((2, 4, 4, L), jnp.int32),
        pltpu.VMEM((L,), jnp.int32),
        pltpu.VMEM((4, L), jnp.int32),
    ],
)


def _group_sc_body(xyz_hbm, norm_hbm,
              nb_out, ctr_out, ng_out, nctr_out, idx_out, idxnew_out,
              xv, yv, zv, nxv, nyv, nzv, xbv, ybv, zbv, xx, dist, d2buf, cent,
              candd, candi, selidx,
              nb_stage, ng_stage, ctr_stage, nctr_stage,
              ccx, ccy, ccz, ccsq, sbuf, stg, mbuf):
    sid = lax.axis_index("s")
    wid = sid * 2 + lax.axis_index("c")
    b = wid % B
    q = wid // B
    bslot = sid % 4

    lane = lax.broadcasted_iota(jnp.int32, (L,), 0)
    lane0 = lane == 0
    zeros16 = jnp.zeros((L,), jnp.int32)
    ones16 = jnp.full((L,), 1, jnp.int32)
    twos16 = jnp.full((L,), 2, jnp.int32)
    inf16 = jnp.full((L,), jnp.inf, jnp.float32)

    pltpu.sync_copy(xyz_hbm.at[b, 0], xv)
    pltpu.sync_copy(xyz_hbm.at[b, 1], yv)
    pltpu.sync_copy(xyz_hbm.at[b, 2], zv)
    pltpu.sync_copy(norm_hbm.at[b, 0], nxv)
    pltpu.sync_copy(norm_hbm.at[b, 1], nyv)
    pltpu.sync_copy(norm_hbm.at[b, 2], nzv)

    def _bf16r(v):
        u = plsc.bitcast(v, jnp.int32)
        r = (u + jnp.int32(0x7FFF)) + ((u >> 16) & jnp.int32(1))
        return plsc.bitcast(r & jnp.int32(-65536), jnp.float32)

    @plsc.parallel_loop(0, N, step=L, unroll=8)
    def _pre(i):
        s = pl.ds(i, L)
        x = xv[s]
        y = yv[s]
        z = zv[s]
        xx[s] = x * x + y * y + z * z
        xbv[s] = _bf16r(x)
        ybv[s] = _bf16r(y)
        zbv[s] = _bf16r(z)
        dist[s] = jnp.full((L,), 1e10, jnp.float32)

    QN = N // 4
    base = q * QN
    neg16 = jnp.full((L,), -jnp.inf, jnp.float32)

    def _fps_step(i, f):
        fv = jnp.full((L,), f, jnp.int32)
        plsc.store_scatter(cent, [jnp.full((L,), i, jnp.int32)], fv,
                           mask=lane0)
        cx = plsc.load_gather(xv, [fv])
        cy = plsc.load_gather(yv, [fv])
        cz = plsc.load_gather(zv, [fv])

        @plsc.parallel_loop(
            0, QN, step=L, unroll=8,
            carry=(jnp.full((L,), -1.0, jnp.float32), zeros16))
        def _chunk(ii, carry):
            vmax, vidx = carry
            s = pl.ds(base + ii, L)
            dx = xv[s] - cx
            dy = yv[s] - cy
            dz = zv[s] - cz
            d = dx * dx + dy * dy + dz * dz
            nd = jnp.minimum(dist[s], d)
            dist[s] = nd
            gt = nd > vmax
            vidx = jnp.where(gt, base + ii + lane, vidx)
            vmax = jnp.where(gt, nd, vmax)
            return vmax, vidx

        vmax, vidx = _chunk
        m = jnp.max(vmax)
        cand = jnp.where(vmax == m, vidx, jnp.full((L,), N, jnp.int32))
        li = jnp.min(cand)
        mkey = plsc.bitcast(jnp.full((L,), m, jnp.float32), jnp.int32)
        stg[...] = jnp.where(lane0, mkey,
                             jnp.where(lane == 1, jnp.full((L,), li,
                                                           jnp.int32),
                                       zeros16))
        par = i & 1
        pltpu.sync_copy(stg, sbuf.at[par, bslot, q])
        plsc.subcore_barrier()
        pltpu.sync_copy(sbuf.at[par, bslot], mbuf)
        vals = plsc.load_gather(mbuf, [lane & 3, zeros16])
        idxs = plsc.load_gather(mbuf, [lane & 3, ones16])
        vmask = jnp.where(lane < 4, vals,
                          jnp.full((L,), jnp.iinfo(jnp.int32).min,
                                   jnp.int32))
        mg = jnp.max(vmask)
        cand2 = jnp.where(vmask == mg, idxs, jnp.full((L,), N, jnp.int32))
        return jnp.min(cand2)

    lax.fori_loop(0, G, _fps_step, jnp.int32(0))

    for h in (0, 1):
        cidx = plsc.load_gather(cent, [q * QC + h * L + lane])
        rows = h * L + lane
        sx = plsc.load_gather(xv, [cidx])
        sy = plsc.load_gather(yv, [cidx])
        sz = plsc.load_gather(zv, [cidx])
        ccx[pl.ds(h * L, L)] = sx
        ccy[pl.ds(h * L, L)] = sy
        ccz[pl.ds(h * L, L)] = sz
        ccsq[pl.ds(h * L, L)] = plsc.load_gather(xx, [cidx])
        plsc.store_scatter(ctr_stage, [rows, zeros16], sx)
        plsc.store_scatter(ctr_stage, [rows, ones16], sy)
        plsc.store_scatter(ctr_stage, [rows, twos16], sz)
        plsc.store_scatter(nctr_stage, [rows, zeros16],
                           plsc.load_gather(nxv, [cidx]))
        plsc.store_scatter(nctr_stage, [rows, ones16],
                           plsc.load_gather(nyv, [cidx]))
        plsc.store_scatter(nctr_stage, [rows, twos16],
                           plsc.load_gather(nzv, [cidx]))

    def _centroid(gl, c):
        glv = jnp.full((L,), gl, jnp.int32)
        cxg = _bf16r(plsc.load_gather(ccx, [glv]))
        cyg = _bf16r(plsc.load_gather(ccy, [glv]))
        czg = _bf16r(plsc.load_gather(ccz, [glv]))
        ccg = plsc.load_gather(ccsq, [glv])

        @plsc.parallel_loop(0, N, step=L, unroll=8, carry=(inf16, inf16))
        def _p1(i, carry):
            vm1, vm2 = carry
            s = pl.ds(i, L)
            t = (cxg * xbv[s] + cyg * ybv[s]) + czg * zbv[s]
            d2 = jnp.maximum((ccg + xx[s]) - 2.0 * t, 0.0)
            d2buf[s] = d2
            isnew = d2 < vm1
            vm2 = jnp.where(isnew, vm1, jnp.minimum(vm2, d2))
            vm1 = jnp.minimum(vm1, d2)
            return vm1, vm2
        vm1, vm2 = _p1
        ubound = jnp.max(vm2)

        @plsc.parallel_loop(0, N, step=L, unroll=8, carry=zeros16)
        def _p2(i, cnt):
            s = pl.ds(i, L)
            d2 = d2buf[s]
            msk = d2 <= ubound
            pos = cnt + plsc.cumsum(msk.astype(jnp.int32)) - 1
            plsc.store_scatter(candd, [pos], d2, mask=msk)
            plsc.store_scatter(candi, [pos], i + lane, mask=msk)
            return cnt + plsc.all_reduce_population_count(msk)
        cntv = _p2
        m_total = jnp.max(cntv)
        plsc.store_scatter(candd, [m_total + lane], inf16)
        nch_c = lax.div(m_total + (L - 1), L)

        def _sel(k, c2):
            def _scan(j, carry):
                vmin, vpos = carry
                s = pl.ds(j * L, L)
                d = candd[s]
                lt = d < vmin
                vpos = jnp.where(lt, j * L + lane, vpos)
                vmin = jnp.where(lt, d, vmin)
                return vmin, vpos
            vmin, vpos = lax.fori_loop(0, nch_c, _scan, (inf16, zeros16))
            mv = jnp.min(vmin)
            pc = jnp.where(vmin == mv, vpos,
                           jnp.full((L,), CAND_CAP + L, jnp.int32))
            p = jnp.min(pc)
            pv = jnp.full((L,), p, jnp.int32)
            iv = plsc.load_gather(candi, [pv])
            plsc.store_scatter(selidx, [glv, jnp.full((L,), k, jnp.int32)],
                               iv, mask=lane0)
            plsc.store_scatter(candd, [pv], inf16, mask=lane0)
            return c2
        lax.fori_loop(0, K, _sel, 0)
        return c
    lax.fori_loop(0, QC, _centroid, 0)

    def _gather(gl, c):
        glv = jnp.full((L,), gl, jnp.int32)
        cxg = plsc.load_gather(ccx, [glv])
        cyg = plsc.load_gather(ccy, [glv])
        czg = plsc.load_gather(ccz, [glv])

        for h in (0, 1):
            ks = h * L + lane
            ii = plsc.load_gather(selidx, [glv, ks])
            gx = plsc.load_gather(xv, [ii])
            gy = plsc.load_gather(yv, [ii])
            gz = plsc.load_gather(zv, [ii])
            plsc.store_scatter(nb_stage, [glv, ks, zeros16], gx - cxg)
            plsc.store_scatter(nb_stage, [glv, ks, ones16], gy - cyg)
            plsc.store_scatter(nb_stage, [glv, ks, twos16], gz - czg)
            plsc.store_scatter(ng_stage, [glv, ks, zeros16],
                               plsc.load_gather(nxv, [ii]))
            plsc.store_scatter(ng_stage, [glv, ks, ones16],
                               plsc.load_gather(nyv, [ii]))
            plsc.store_scatter(ng_stage, [glv, ks, twos16],
                               plsc.load_gather(nzv, [ii]))
        return c
    lax.fori_loop(0, QC, _gather, 0)

    gsl = pl.ds(q * QC, QC)
    pltpu.sync_copy(nb_stage, nb_out.at[b, gsl])
    pltpu.sync_copy(ctr_stage, ctr_out.at[b, gsl])
    pltpu.sync_copy(ng_stage, ng_out.at[b, gsl])
    pltpu.sync_copy(nctr_stage, nctr_out.at[b, gsl])
    pltpu.sync_copy(selidx, idx_out.at[b, gsl])
    pltpu.sync_copy(cent.at[gsl], idxnew_out.at[b, gsl])


_group_sc = pl.kernel(_group_sc_body, **_KERNEL_KWARGS)


def kernel(xyz, norm):
    xyz_t = jnp.transpose(xyz, (0, 2, 1))
    norm_t = jnp.transpose(norm, (0, 2, 1))
    nb, ctr, ng, nctr, idx, idx_new = _group_sc(xyz_t, norm_t)
    return (nb, ctr, ng, nctr, idx, idx_new)


# speedup vs baseline: 10.1933x; 1.0381x over previous
import functools

import jax
import jax.numpy as jnp
from jax import lax
from jax.experimental import pallas as pl
from jax.experimental.pallas import tpu as pltpu
from jax.experimental.pallas import tpu_sc as plsc

B = 8
N = 8192
G = 128
K = 32
L = 16
NCH = N // L
QC = G // 4
CAND_CAP = 2048

_mesh = plsc.VectorSubcoreMesh(core_axis_name="c", subcore_axis_name="s")

_KERNEL_KWARGS = dict(
    out_type=[
        jax.ShapeDtypeStruct((B, G, K, 3), jnp.float32),
        jax.ShapeDtypeStruct((B, G, 3), jnp.float32),
        jax.ShapeDtypeStruct((B, G, K, 3), jnp.float32),
        jax.ShapeDtypeStruct((B, G, 3), jnp.float32),
        jax.ShapeDtypeStruct((B, G, K), jnp.int32),
        jax.ShapeDtypeStruct((B, G), jnp.int32),
    ],
    mesh=_mesh,
    compiler_params=pltpu.CompilerParams(
        needs_layout_passes=False, use_tc_tiling_on_sc=False),
    scratch_types=[
        pltpu.VMEM((N,), jnp.float32),
        pltpu.VMEM((N,), jnp.float32),
        pltpu.VMEM((N,), jnp.float32),
        pltpu.VMEM((N,), jnp.float32),
        pltpu.VMEM((N,), jnp.float32),
        pltpu.VMEM((N,), jnp.float32),
        pltpu.VMEM((N,), jnp.float32),
        pltpu.VMEM((N,), jnp.float32),
        pltpu.VMEM((N,), jnp.float32),
        pltpu.VMEM((N,), jnp.float32),
        pltpu.VMEM((N,), jnp.float32),
        pltpu.VMEM((N,), jnp.float32),
        pltpu.VMEM((G,), jnp.int32),
        pltpu.VMEM((CAND_CAP + L,), jnp.float32),
        pltpu.VMEM((CAND_CAP + L,), jnp.int32),
        pltpu.VMEM((QC, K), jnp.int32),
        pltpu.VMEM((QC, K, 3), jnp.float32),
        pltpu.VMEM((QC, K, 3), jnp.float32),
        pltpu.VMEM((QC, 3), jnp.float32),
        pltpu.VMEM((QC, 3), jnp.float32),
        pltpu.VMEM((QC,), jnp.float32),
        pltpu.VMEM((QC,), jnp.float32),
        pltpu.VMEM((QC,), jnp.float32),
        pltpu.VMEM((QC,), jnp.float32),
        pltpu.VMEM_SHARED((2, 4, 4, L), jnp.int32),
        pltpu.VMEM((L,), jnp.int32),
        pltpu.VMEM((4, L), jnp.int32),
    ],
)


def _group_sc_body(xyz_hbm, norm_hbm,
              nb_out, ctr_out, ng_out, nctr_out, idx_out, idxnew_out,
              xv, yv, zv, nxv, nyv, nzv, xbv, ybv, zbv, xx, dist, d2buf, cent,
              candd, candi, selidx,
              nb_stage, ng_stage, ctr_stage, nctr_stage,
              ccx, ccy, ccz, ccsq, sbuf, stg, mbuf):
    sid = lax.axis_index("s")
    wid = sid * 2 + lax.axis_index("c")
    b = wid % B
    q = wid // B
    bslot = sid % 4

    lane = lax.broadcasted_iota(jnp.int32, (L,), 0)
    lane0 = lane == 0
    zeros16 = jnp.zeros((L,), jnp.int32)
    ones16 = jnp.full((L,), 1, jnp.int32)
    twos16 = jnp.full((L,), 2, jnp.int32)
    inf16 = jnp.full((L,), jnp.inf, jnp.float32)

    pltpu.sync_copy(xyz_hbm.at[b, 0], xv)
    pltpu.sync_copy(xyz_hbm.at[b, 1], yv)
    pltpu.sync_copy(xyz_hbm.at[b, 2], zv)
    pltpu.sync_copy(norm_hbm.at[b, 0], nxv)
    pltpu.sync_copy(norm_hbm.at[b, 1], nyv)
    pltpu.sync_copy(norm_hbm.at[b, 2], nzv)

    def _bf16r(v):
        u = plsc.bitcast(v, jnp.int32)
        r = (u + jnp.int32(0x7FFF)) + ((u >> 16) & jnp.int32(1))
        return plsc.bitcast(r & jnp.int32(-65536), jnp.float32)

    @plsc.parallel_loop(0, N, step=L, unroll=8)
    def _pre(i):
        s = pl.ds(i, L)
        x = xv[s]
        y = yv[s]
        z = zv[s]
        xx[s] = x * x + y * y + z * z
        xbv[s] = _bf16r(x)
        ybv[s] = _bf16r(y)
        zbv[s] = _bf16r(z)
        dist[s] = jnp.full((L,), 1e10, jnp.float32)

    QN = N // 4
    base = q * QN
    neg16 = jnp.full((L,), -jnp.inf, jnp.float32)

    def _fps_step(i, f):
        fv = jnp.full((L,), f, jnp.int32)
        plsc.store_scatter(cent, [jnp.full((L,), i, jnp.int32)], fv,
                           mask=lane0)
        cx = plsc.load_gather(xv, [fv])
        cy = plsc.load_gather(yv, [fv])
        cz = plsc.load_gather(zv, [fv])

        @plsc.parallel_loop(
            0, QN, step=L, unroll=8,
            carry=(jnp.full((L,), -1.0, jnp.float32), zeros16))
        def _chunk(ii, carry):
            vmax, vidx = carry
            s = pl.ds(base + ii, L)
            dx = xv[s] - cx
            dy = yv[s] - cy
            dz = zv[s] - cz
            d = dx * dx + dy * dy + dz * dz
            nd = jnp.minimum(dist[s], d)
            dist[s] = nd
            gt = nd > vmax
            vidx = jnp.where(gt, base + ii + lane, vidx)
            vmax = jnp.where(gt, nd, vmax)
            return vmax, vidx

        vmax, vidx = _chunk
        m = jnp.max(vmax)
        cand = jnp.where(vmax == m, vidx, jnp.full((L,), N, jnp.int32))
        li = jnp.min(cand)
        mkey = plsc.bitcast(jnp.full((L,), m, jnp.float32), jnp.int32)
        stg[...] = jnp.where(lane0, mkey,
                             jnp.where(lane == 1, jnp.full((L,), li,
                                                           jnp.int32),
                                       zeros16))
        par = i & 1
        pltpu.sync_copy(stg, sbuf.at[par, bslot, q])
        plsc.subcore_barrier()
        pltpu.sync_copy(sbuf.at[par, bslot], mbuf)
        vals = plsc.load_gather(mbuf, [lane & 3, zeros16])
        idxs = plsc.load_gather(mbuf, [lane & 3, ones16])
        vmask = jnp.where(lane < 4, vals,
                          jnp.full((L,), jnp.iinfo(jnp.int32).min,
                                   jnp.int32))
        mg = jnp.max(vmask)
        cand2 = jnp.where(vmask == mg, idxs, jnp.full((L,), N, jnp.int32))
        return jnp.min(cand2)

    lax.fori_loop(0, G, _fps_step, jnp.int32(0))

    for h in (0, 1):
        cidx = plsc.load_gather(cent, [q * QC + h * L + lane])
        rows = h * L + lane
        sx = plsc.load_gather(xv, [cidx])
        sy = plsc.load_gather(yv, [cidx])
        sz = plsc.load_gather(zv, [cidx])
        ccx[pl.ds(h * L, L)] = sx
        ccy[pl.ds(h * L, L)] = sy
        ccz[pl.ds(h * L, L)] = sz
        ccsq[pl.ds(h * L, L)] = plsc.load_gather(xx, [cidx])
        plsc.store_scatter(ctr_stage, [rows, zeros16], sx)
        plsc.store_scatter(ctr_stage, [rows, ones16], sy)
        plsc.store_scatter(ctr_stage, [rows, twos16], sz)
        plsc.store_scatter(nctr_stage, [rows, zeros16],
                           plsc.load_gather(nxv, [cidx]))
        plsc.store_scatter(nctr_stage, [rows, ones16],
                           plsc.load_gather(nyv, [cidx]))
        plsc.store_scatter(nctr_stage, [rows, twos16],
                           plsc.load_gather(nzv, [cidx]))

    def _centroid(gl, c):
        glv = jnp.full((L,), gl, jnp.int32)
        cxg = 2.0 * _bf16r(plsc.load_gather(ccx, [glv]))
        cyg = 2.0 * _bf16r(plsc.load_gather(ccy, [glv]))
        czg = 2.0 * _bf16r(plsc.load_gather(ccz, [glv]))
        ccg = plsc.load_gather(ccsq, [glv])

        @plsc.parallel_loop(0, N, step=L, unroll=8, carry=(inf16, inf16))
        def _p1(i, carry):
            vm1, vm2 = carry
            s = pl.ds(i, L)
            t = (cxg * xbv[s] + cyg * ybv[s]) + czg * zbv[s]
            d2 = jnp.maximum((ccg + xx[s]) - t, 0.0)
            d2buf[s] = d2
            isnew = d2 < vm1
            vm2 = jnp.where(isnew, vm1, jnp.minimum(vm2, d2))
            vm1 = jnp.minimum(vm1, d2)
            return vm1, vm2
        vm1, vm2 = _p1
        ubound = jnp.max(vm2)

        @plsc.parallel_loop(0, N, step=L, unroll=8, carry=zeros16)
        def _p2(i, cnt):
            s = pl.ds(i, L)
            d2 = d2buf[s]
            msk = d2 <= ubound
            pos = cnt + plsc.cumsum(msk.astype(jnp.int32)) - 1
            plsc.store_scatter(candd, [pos], d2, mask=msk)
            plsc.store_scatter(candi, [pos], i + lane, mask=msk)
            return cnt + plsc.all_reduce_population_count(msk)
        cntv = _p2
        m_total = jnp.max(cntv)
        plsc.store_scatter(candd, [m_total + lane], inf16)
        nch_c = lax.div(m_total + (L - 1), L)

        def _sel(k, c2):
            @plsc.parallel_loop(0, nch_c * L, step=L, unroll=4,
                                carry=(inf16, zeros16))
            def _scan(j, carry):
                vmin, vpos = carry
                s = pl.ds(j, L)
                d = candd[s]
                lt = d < vmin
                vpos = jnp.where(lt, j + lane, vpos)
                vmin = jnp.where(lt, d, vmin)
                return vmin, vpos
            vmin, vpos = _scan
            mv = jnp.min(vmin)
            pc = jnp.where(vmin == mv, vpos,
                           jnp.full((L,), CAND_CAP + L, jnp.int32))
            p = jnp.min(pc)
            pv = jnp.full((L,), p, jnp.int32)
            iv = plsc.load_gather(candi, [pv])
            plsc.store_scatter(selidx, [glv, jnp.full((L,), k, jnp.int32)],
                               iv, mask=lane0)
            plsc.store_scatter(candd, [pv], inf16, mask=lane0)
            return c2
        lax.fori_loop(0, K, _sel, 0)
        return c
    lax.fori_loop(0, QC, _centroid, 0)

    def _gather(gl, c):
        glv = jnp.full((L,), gl, jnp.int32)
        cxg = plsc.load_gather(ccx, [glv])
        cyg = plsc.load_gather(ccy, [glv])
        czg = plsc.load_gather(ccz, [glv])

        for h in (0, 1):
            ks = h * L + lane
            ii = plsc.load_gather(selidx, [glv, ks])
            gx = plsc.load_gather(xv, [ii])
            gy = plsc.load_gather(yv, [ii])
            gz = plsc.load_gather(zv, [ii])
            plsc.store_scatter(nb_stage, [glv, ks, zeros16], gx - cxg)
            plsc.store_scatter(nb_stage, [glv, ks, ones16], gy - cyg)
            plsc.store_scatter(nb_stage, [glv, ks, twos16], gz - czg)
            plsc.store_scatter(ng_stage, [glv, ks, zeros16],
                               plsc.load_gather(nxv, [ii]))
            plsc.store_scatter(ng_stage, [glv, ks, ones16],
                               plsc.load_gather(nyv, [ii]))
            plsc.store_scatter(ng_stage, [glv, ks, twos16],
                               plsc.load_gather(nzv, [ii]))
        return c
    lax.fori_loop(0, QC, _gather, 0)

    gsl = pl.ds(q * QC, QC)
    pltpu.sync_copy(nb_stage, nb_out.at[b, gsl])
    pltpu.sync_copy(ctr_stage, ctr_out.at[b, gsl])
    pltpu.sync_copy(ng_stage, ng_out.at[b, gsl])
    pltpu.sync_copy(nctr_stage, nctr_out.at[b, gsl])
    pltpu.sync_copy(selidx, idx_out.at[b, gsl])
    pltpu.sync_copy(cent.at[gsl], idxnew_out.at[b, gsl])


_group_sc = pl.kernel(_group_sc_body, **_KERNEL_KWARGS)


def kernel(xyz, norm):
    xyz_t = jnp.transpose(xyz, (0, 2, 1))
    norm_t = jnp.transpose(norm, (0, 2, 1))
    nb, ctr, ng, nctr, idx, idx_new = _group_sc(xyz_t, norm_t)
    return (nb, ctr, ng, nctr, idx, idx_new)
